# Initial kernel scaffold; baseline (speedup 1.0000x reference)
import jax
import jax.numpy as jnp
from jax.experimental import pallas as pl


def kernel(disp_param, numbers, disp_param0):
    raise NotImplementedError("write your pallas kernel here")


# speedup vs baseline: 1.1056x; 1.1056x over previous
import functools

import jax
import jax.numpy as jnp
from jax import lax
from jax.experimental import pallas as pl
from jax.experimental.pallas import tpu as pltpu
from jax.experimental.pallas import tpu_sc as plsc

_NUM_CORES = 2
_NUM_SUBCORES = 16
_NUM_WORKERS = _NUM_CORES * _NUM_SUBCORES
_LANES = 16

_CHUNK_ROWS = 8000


def _sc_disp_param(n_rows):
  assert n_rows % _CHUNK_ROWS == 0
  n_chunks = n_rows // _CHUNK_ROWS
  rounds = -(-n_chunks // _NUM_WORKERS)
  chunk_f = 2 * _CHUNK_ROWS

  mesh = plsc.VectorSubcoreMesh(
      core_axis_name="c", subcore_axis_name="s",
      num_cores=_NUM_CORES, num_subcores=_NUM_SUBCORES)

  @functools.partial(
      pl.kernel,
      out_type=jax.ShapeDtypeStruct((2 * n_rows,), jnp.float32),
      mesh=mesh,
      scratch_types=[
          pltpu.VMEM((_CHUNK_ROWS,), jnp.int32),
          pltpu.VMEM((chunk_f,), jnp.float32),
          pltpu.VMEM((chunk_f,), jnp.float32),
          pltpu.VMEM((87, 2), jnp.float32),
      ],
      compiler_params=pltpu.CompilerParams(needs_layout_passes=False),
  )
  def body(disp_hbm, nums_hbm, tab_hbm, out_hbm, nums_v, in_v, out_v, tab_v):
    w = lax.axis_index("s") * _NUM_CORES + lax.axis_index("c")
    pltpu.sync_copy(tab_hbm, tab_v)

    for k in range(rounds):
      cid = w + _NUM_WORKERS * k

      @pl.when(cid < n_chunks)
      def _():
        row0 = cid * _CHUNK_ROWS
        pltpu.sync_copy(nums_hbm.at[pl.ds(row0, _CHUNK_ROWS)], nums_v)
        pltpu.sync_copy(disp_hbm.at[pl.ds(2 * row0, chunk_f)], in_v)

        def step(j, carry):
          bf = j * _LANES
          iota = lax.iota(jnp.int32, _LANES)
          one = jnp.full((_LANES,), 1, jnp.int32)
          eidx = lax.shift_right_logical(iota, one)
          parity = jnp.bitwise_and(iota, one)
          nidx = eidx + jnp.full((_LANES,), j * (_LANES // 2), jnp.int32)
          nums16 = plsc.load_gather(nums_v, [nidx])
          g = plsc.load_gather(tab_v, [nums16, parity])
          x = in_v[pl.ds(bf, _LANES)]
          lo = jnp.full((_LANES,), -4.0, jnp.float32)
          hi = jnp.full((_LANES,), 4.0, jnp.float32)
          m = jnp.exp(jnp.maximum(jnp.minimum(x, hi), lo))
          out_v[pl.ds(bf, _LANES)] = g * m
          return carry

        lax.fori_loop(0, chunk_f // _LANES, step, 0)
        pltpu.sync_copy(out_v, out_hbm.at[pl.ds(2 * row0, chunk_f)])

  return body


def kernel(disp_param, numbers, disp_param0):
  n_rows = disp_param.shape[0]
  fn = _sc_disp_param(n_rows)
  out_flat = fn(disp_param.reshape(-1), numbers, disp_param0)
  return out_flat.reshape(n_rows, 2)
